# Initial kernel scaffold; baseline (speedup 1.0000x reference)
import jax
import jax.numpy as jnp
from jax.experimental import pallas as pl


def kernel(x, adjs, W_l1, W_r1, b1, W_l2, W_r2, b2, W_lo, W_ro, bo):
    raise NotImplementedError("write your pallas kernel here")


# speedup vs baseline: 4.9055x; 4.9055x over previous
import functools

import jax
import jax.numpy as jnp
from jax import lax
from jax.experimental import pallas as pl
from jax.experimental.pallas import tpu as pltpu
from jax.experimental.pallas import tpu_sc as plsc

N = 10000
E = 320000
F = 128
C_OUT = 64

NC = 2
NS = 16
NW = NC * NS
NPAD = 10112
ROWS_PS = NPAD // NS
EPW = E // NW
CHUNK = 80
NCHUNK = EPW // CHUNK
DEG_W = 16


def _make_sc_segsum(d, with_deg):
  mesh = plsc.VectorSubcoreMesh(core_axis_name="c", subcore_axis_name="s",
                                num_cores=NC, num_subcores=NS)
  params = pltpu.CompilerParams(use_tc_tiling_on_sc=False)

  out_type = [jax.ShapeDtypeStruct((NC * NPAD, d), jnp.float32)]
  scratch = [
      pltpu.VMEM((CHUNK,), jnp.int32),
      pltpu.VMEM((CHUNK,), jnp.int32),
      pltpu.VMEM((CHUNK, d), jnp.float32),
      pltpu.VMEM_SHARED((NPAD, d), jnp.float32),
      pltpu.SemaphoreType.DMA,
  ]
  if with_deg:
    out_type.append(jax.ShapeDtypeStruct((NC * NPAD, DEG_W), jnp.float32))
    scratch += [
        pltpu.VMEM((CHUNK, DEG_W), jnp.float32),
        pltpu.VMEM((CHUNK, DEG_W), jnp.float32),
        pltpu.VMEM_SHARED((NPAD, DEG_W), jnp.float32),
    ]

  chunks = [(k * CHUNK, CHUNK) for k in range(ROWS_PS // CHUNK)]
  if ROWS_PS % CHUNK:
    chunks.append((ROWS_PS - ROWS_PS % CHUNK, ROWS_PS % CHUNK))

  def body(*refs):
    if with_deg:
      (p_hbm, src_hbm, dst_hbm, zrow_hbm, zdeg_hbm, ones_hbm,
       acc_out, deg_out, srcv, dstv, rows, acc_sh, sem,
       onesv, degv, deg_sh) = refs
    else:
      (p_hbm, src_hbm, dst_hbm, zrow_hbm,
       acc_out, srcv, dstv, rows, acc_sh, sem) = refs

    c = lax.axis_index("c")
    s = lax.axis_index("s")
    wid = c * NS + s
    roff = s * ROWS_PS

    pltpu.sync_copy(zrow_hbm, rows)
    for off, sz in chunks:
      pltpu.sync_copy(rows.at[pl.ds(0, sz)], acc_sh.at[pl.ds(roff + off, sz)])
    if with_deg:
      pltpu.sync_copy(zdeg_hbm, degv)
      for off, sz in chunks:
        pltpu.sync_copy(degv.at[pl.ds(0, sz)],
                        deg_sh.at[pl.ds(roff + off, sz)])
      pltpu.sync_copy(ones_hbm, onesv)
    plsc.subcore_barrier()

    ebase = wid * EPW

    def step(i, carry):
      base = ebase + i * CHUNK
      pltpu.sync_copy(src_hbm.at[pl.ds(base, CHUNK)], srcv)
      pltpu.sync_copy(dst_hbm.at[pl.ds(base, CHUNK)], dstv)
      pltpu.async_copy(p_hbm.at[srcv], rows, sem).wait()
      pltpu.sync_copy(rows, acc_sh.at[dstv], add=True)
      if with_deg:
        pltpu.sync_copy(onesv, deg_sh.at[dstv], add=True)
      return carry

    lax.fori_loop(0, NCHUNK, step, 0)
    plsc.subcore_barrier()

    for off, sz in chunks:
      pltpu.sync_copy(acc_sh.at[pl.ds(roff + off, sz)], rows.at[pl.ds(0, sz)])
      pltpu.sync_copy(rows.at[pl.ds(0, sz)],
                      acc_out.at[pl.ds(c * NPAD + roff + off, sz)])
    if with_deg:
      for off, sz in chunks:
        pltpu.sync_copy(deg_sh.at[pl.ds(roff + off, sz)],
                        degv.at[pl.ds(0, sz)])
        pltpu.sync_copy(degv.at[pl.ds(0, sz)],
                        deg_out.at[pl.ds(c * NPAD + roff + off, sz)])

  return pl.kernel(body, out_type=out_type, mesh=mesh, scratch_types=scratch,
                   compiler_params=params)


_R = 1000


def _tc_in(x, W_l, W_r, b):
  def body(x_ref, wl_ref, wr_ref, b_ref, p_ref, r_ref):
    xb = x_ref[...]
    p_ref[...] = jnp.dot(xb, wl_ref[...], preferred_element_type=jnp.float32)
    r_ref[...] = (jnp.dot(xb, wr_ref[...], preferred_element_type=jnp.float32)
                  + b_ref[...])

  d = W_l.shape[1]
  return pl.pallas_call(
      body,
      grid=(N // _R,),
      in_specs=[
          pl.BlockSpec((_R, F), lambda i: (i, 0)),
          pl.BlockSpec((F, d), lambda i: (0, 0)),
          pl.BlockSpec((F, d), lambda i: (0, 0)),
          pl.BlockSpec((1, d), lambda i: (0, 0)),
      ],
      out_specs=[
          pl.BlockSpec((_R, d), lambda i: (i, 0)),
          pl.BlockSpec((_R, d), lambda i: (i, 0)),
      ],
      out_shape=[jax.ShapeDtypeStruct((N, d), jnp.float32)] * 2,
  )(x, W_l, W_r, b.reshape(1, d))


def _tc_combine(acc, deg, r_prev, W_l, W_r, b):
  d_in = acc.shape[-1]
  d_out = W_l.shape[1]

  def body(acc_ref, deg_ref, r_ref, wl_ref, wr_ref, b_ref, p_ref, rn_ref):
    agg = acc_ref[0] + acc_ref[1]
    dg = deg_ref[0, :, 0:1] + deg_ref[1, :, 0:1]
    inv = 1.0 / jnp.maximum(dg, 1.0)
    h = jnp.maximum(agg * inv + r_ref[...], 0.0)
    p_ref[...] = jnp.dot(h, wl_ref[...], preferred_element_type=jnp.float32)
    rn_ref[...] = (jnp.dot(h, wr_ref[...], preferred_element_type=jnp.float32)
                   + b_ref[...])

  return pl.pallas_call(
      body,
      grid=(N // _R,),
      in_specs=[
          pl.BlockSpec((NC, _R, d_in), lambda i: (0, i, 0)),
          pl.BlockSpec((NC, _R, DEG_W), lambda i: (0, i, 0)),
          pl.BlockSpec((_R, d_in), lambda i: (i, 0)),
          pl.BlockSpec((F, d_out), lambda i: (0, 0)),
          pl.BlockSpec((F, d_out), lambda i: (0, 0)),
          pl.BlockSpec((1, d_out), lambda i: (0, 0)),
      ],
      out_specs=[
          pl.BlockSpec((_R, d_out), lambda i: (i, 0)),
          pl.BlockSpec((_R, d_out), lambda i: (i, 0)),
      ],
      out_shape=[jax.ShapeDtypeStruct((N, d_out), jnp.float32)] * 2,
  )(acc, deg, r_prev, W_l, W_r, b.reshape(1, d_out))


def _tc_final(acc, deg, r_prev):
  d = acc.shape[-1]

  def body(acc_ref, deg_ref, r_ref, o_ref):
    agg = acc_ref[0] + acc_ref[1]
    dg = deg_ref[0, :, 0:1] + deg_ref[1, :, 0:1]
    inv = 1.0 / jnp.maximum(dg, 1.0)
    o_ref[...] = agg * inv + r_ref[...]

  return pl.pallas_call(
      body,
      grid=(N // _R,),
      in_specs=[
          pl.BlockSpec((NC, _R, d), lambda i: (0, i, 0)),
          pl.BlockSpec((NC, _R, DEG_W), lambda i: (0, i, 0)),
          pl.BlockSpec((_R, d), lambda i: (i, 0)),
      ],
      out_specs=pl.BlockSpec((_R, d), lambda i: (i, 0)),
      out_shape=jax.ShapeDtypeStruct((N, d), jnp.float32),
  )(acc, deg, r_prev)


@jax.jit
def kernel(x, adjs, W_l1, W_r1, b1, W_l2, W_r2, b2, W_lo, W_ro, bo):
  src = adjs[0].astype(jnp.int32)
  dst = adjs[1].astype(jnp.int32)

  zrow128 = jnp.zeros((CHUNK, F), jnp.float32)
  zrow64 = jnp.zeros((CHUNK, C_OUT), jnp.float32)
  zdeg = jnp.zeros((CHUNK, DEG_W), jnp.float32)
  ones = jnp.ones((CHUNK, DEG_W), jnp.float32)

  sc_128_deg = _make_sc_segsum(F, with_deg=True)
  sc_128 = _make_sc_segsum(F, with_deg=False)
  sc_64 = _make_sc_segsum(C_OUT, with_deg=False)

  p1, r1 = _tc_in(x, W_l1, W_r1, b1)
  acc1, deg = sc_128_deg(p1, src, dst, zrow128, zdeg, ones)
  acc1 = acc1.reshape(NC, NPAD, F)
  deg = deg.reshape(NC, NPAD, DEG_W)
  p2, r2 = _tc_combine(acc1, deg, r1, W_l2, W_r2, b2)
  (acc2,) = sc_128(p2, src, dst, zrow128)
  acc2 = acc2.reshape(NC, NPAD, F)
  p3, r3 = _tc_combine(acc2, deg, r2, W_lo, W_ro, bo)
  (acc3,) = sc_64(p3, src, dst, zrow64)
  acc3 = acc3.reshape(NC, NPAD, C_OUT)
  return _tc_final(acc3, deg, r3)


# speedup vs baseline: 12.8446x; 2.6184x over previous
import functools

import jax
import jax.numpy as jnp
from jax import lax
from jax.experimental import pallas as pl
from jax.experimental.pallas import tpu as pltpu
from jax.experimental.pallas import tpu_sc as plsc

N = 10000
E = 320000
F = 128
C_OUT = 64

NC = 2
NS = 16
NW = NC * NS
NPAD = 10112
ROWS_PS = NPAD // NS
EPW = E // NW
CHUNK = 80
NCHUNK = EPW // CHUNK
IG = 5
DEG_W = 16


def _make_sc_segsum(d, with_deg):
  mesh = plsc.VectorSubcoreMesh(core_axis_name="c", subcore_axis_name="s",
                                num_cores=NC, num_subcores=NS)
  params = pltpu.CompilerParams(use_tc_tiling_on_sc=False)

  out_type = [jax.ShapeDtypeStruct((NC * NPAD, d), jnp.float32)]
  scratch = [
      pltpu.VMEM((3, IG, CHUNK), jnp.int32),
      pltpu.VMEM((3, IG, CHUNK), jnp.int32),
      pltpu.VMEM((3, CHUNK, d), jnp.float32),
      pltpu.VMEM_SHARED((NPAD, d), jnp.float32),
      pltpu.SemaphoreType.DMA,
      pltpu.SemaphoreType.DMA,
      pltpu.SemaphoreType.DMA,
  ]
  if with_deg:
    out_type.append(jax.ShapeDtypeStruct((NC * NPAD, DEG_W), jnp.float32))
    scratch += [
        pltpu.VMEM((CHUNK, DEG_W), jnp.float32),
        pltpu.VMEM((CHUNK, DEG_W), jnp.float32),
        pltpu.VMEM_SHARED((NPAD, DEG_W), jnp.float32),
        pltpu.SemaphoreType.DMA,
    ]

  chunks = [(k * CHUNK, CHUNK) for k in range(ROWS_PS // CHUNK)]
  if ROWS_PS % CHUNK:
    chunks.append((ROWS_PS - ROWS_PS % CHUNK, ROWS_PS % CHUNK))

  def body(*refs):
    if with_deg:
      (p_hbm, src_hbm, dst_hbm, zrow_hbm, zdeg_hbm, ones_hbm,
       acc_out, deg_out, srcb, dstb, rows, acc_sh, gsem, ssem, isem,
       onesv, degv, deg_sh, dsem) = refs
    else:
      (p_hbm, src_hbm, dst_hbm, zrow_hbm,
       acc_out, srcb, dstb, rows, acc_sh, gsem, ssem, isem) = refs

    c = lax.axis_index("c")
    s = lax.axis_index("s")
    wid = c * NS + s
    roff = s * ROWS_PS
    ibase = wid * NCHUNK

    def fire_idx(g, slot):
      pltpu.async_copy(src_hbm.at[pl.ds(ibase + g * IG, IG)],
                       srcb.at[slot], isem)
      pltpu.async_copy(dst_hbm.at[pl.ds(ibase + g * IG, IG)],
                       dstb.at[slot], isem)

    def wait_idx():
      pltpu.make_async_copy(src_hbm.at[pl.ds(ibase, IG)],
                            srcb.at[0], isem).wait()
      pltpu.make_async_copy(dst_hbm.at[pl.ds(ibase, IG)],
                            dstb.at[0], isem).wait()

    fire_idx(0, 0)
    fire_idx(1, 1)

    pltpu.sync_copy(zrow_hbm, rows.at[0])
    for off, sz in chunks:
      pltpu.async_copy(rows.at[0, pl.ds(0, sz)],
                       acc_sh.at[pl.ds(roff + off, sz)], ssem)
    if with_deg:
      pltpu.sync_copy(zdeg_hbm, degv)
      for off, sz in chunks:
        pltpu.async_copy(degv.at[pl.ds(0, sz)],
                         deg_sh.at[pl.ds(roff + off, sz)], dsem)
      pltpu.sync_copy(ones_hbm, onesv)
    for off, sz in chunks:
      pltpu.make_async_copy(rows.at[0, pl.ds(0, sz)],
                            acc_sh.at[pl.ds(roff + off, sz)], ssem).wait()
      if with_deg:
        pltpu.make_async_copy(degv.at[pl.ds(0, sz)],
                              deg_sh.at[pl.ds(roff + off, sz)], dsem).wait()
    wait_idx()
    plsc.subcore_barrier()

    def fire_gather(gs, j, slot):
      pltpu.async_copy(p_hbm.at[srcb.at[gs, j]], rows.at[slot], gsem)

    def wait_gather(slot):
      pltpu.make_async_copy(p_hbm.at[srcb.at[0, 0]], rows.at[slot],
                            gsem).wait()

    def fire_scatter(gs, j, slot):
      pltpu.async_copy(rows.at[slot], acc_sh.at[dstb.at[gs, j]], ssem,
                       add=True)
      if with_deg:
        pltpu.async_copy(onesv, deg_sh.at[dstb.at[gs, j]], dsem, add=True)

    def drain_scatter():
      pltpu.make_async_copy(rows.at[0], acc_sh.at[dstb.at[0, 0]],
                            ssem).wait()
      if with_deg:
        pltpu.make_async_copy(onesv, deg_sh.at[dstb.at[0, 0]], dsem).wait()

    fire_gather(0, 0, 0)
    fire_gather(0, 1, 1)
    wait_gather(0)
    fire_scatter(0, 0, 0)
    fire_gather(0, 2, 2)
    wait_gather(1)
    fire_scatter(0, 1, 1)

    def step(i, carry):
      g = lax.div(i, IG)
      j = lax.rem(i, IG)
      gs = lax.rem(g, 3)
      slot = lax.rem(i, 3)
      prev = lax.rem(i - 1, 3)
      gprev = lax.rem(lax.div(i - 1, IG), 3)
      jprev = lax.rem(i - 1, IG)

      @pl.when(j == 0)
      def _():
        wait_idx()

      @pl.when(jnp.logical_and(j == 0, i + IG < NCHUNK))
      def _():
        fire_idx(g + 1, lax.rem(g + 1, 3))

      drain_scatter()
      fire_gather(gs, j, slot)
      wait_gather(prev)
      fire_scatter(gprev, jprev, prev)
      return carry

    lax.fori_loop(3, NCHUNK, step, 0)

    wait_gather((NCHUNK - 1) % 3)
    fire_scatter((NCHUNK - 1) // IG % 3, (NCHUNK - 1) % IG, (NCHUNK - 1) % 3)
    for _ in range(3):
      drain_scatter()
    plsc.subcore_barrier()

    for off, sz in chunks:
      pltpu.sync_copy(acc_sh.at[pl.ds(roff + off, sz)],
                      rows.at[0, pl.ds(0, sz)])
      pltpu.async_copy(rows.at[0, pl.ds(0, sz)],
                       acc_out.at[pl.ds(c * NPAD + roff + off, sz)], gsem)
      pltpu.make_async_copy(rows.at[0, pl.ds(0, sz)],
                            acc_out.at[pl.ds(c * NPAD + roff + off, sz)],
                            gsem).wait()
    if with_deg:
      for off, sz in chunks:
        pltpu.sync_copy(deg_sh.at[pl.ds(roff + off, sz)],
                        degv.at[pl.ds(0, sz)])
        pltpu.async_copy(degv.at[pl.ds(0, sz)],
                         deg_out.at[pl.ds(c * NPAD + roff + off, sz)], gsem)
        pltpu.make_async_copy(degv.at[pl.ds(0, sz)],
                              deg_out.at[pl.ds(c * NPAD + roff + off, sz)],
                              gsem).wait()

  return pl.kernel(body, out_type=out_type, mesh=mesh, scratch_types=scratch,
                   compiler_params=params)


_R = 1000


def _tc_in(x, W_l, W_r, b):
  def body(x_ref, wl_ref, wr_ref, b_ref, p_ref, r_ref):
    xb = x_ref[...]
    p_ref[...] = jnp.dot(xb, wl_ref[...], preferred_element_type=jnp.float32)
    r_ref[...] = (jnp.dot(xb, wr_ref[...], preferred_element_type=jnp.float32)
                  + b_ref[...])

  d = W_l.shape[1]
  return pl.pallas_call(
      body,
      grid=(N // _R,),
      in_specs=[
          pl.BlockSpec((_R, F), lambda i: (i, 0)),
          pl.BlockSpec((F, d), lambda i: (0, 0)),
          pl.BlockSpec((F, d), lambda i: (0, 0)),
          pl.BlockSpec((1, d), lambda i: (0, 0)),
      ],
      out_specs=[
          pl.BlockSpec((_R, d), lambda i: (i, 0)),
          pl.BlockSpec((_R, d), lambda i: (i, 0)),
      ],
      out_shape=[jax.ShapeDtypeStruct((N, d), jnp.float32)] * 2,
  )(x, W_l, W_r, b.reshape(1, d))


def _tc_combine(acc, deg, r_prev, W_l, W_r, b):
  d_in = acc.shape[-1]
  d_out = W_l.shape[1]

  def body(acc_ref, deg_ref, r_ref, wl_ref, wr_ref, b_ref, p_ref, rn_ref):
    agg = acc_ref[0] + acc_ref[1]
    dg = deg_ref[0, :, 0:1] + deg_ref[1, :, 0:1]
    inv = 1.0 / jnp.maximum(dg, 1.0)
    h = jnp.maximum(agg * inv + r_ref[...], 0.0)
    p_ref[...] = jnp.dot(h, wl_ref[...], preferred_element_type=jnp.float32)
    rn_ref[...] = (jnp.dot(h, wr_ref[...], preferred_element_type=jnp.float32)
                   + b_ref[...])

  return pl.pallas_call(
      body,
      grid=(N // _R,),
      in_specs=[
          pl.BlockSpec((NC, _R, d_in), lambda i: (0, i, 0)),
          pl.BlockSpec((NC, _R, DEG_W), lambda i: (0, i, 0)),
          pl.BlockSpec((_R, d_in), lambda i: (i, 0)),
          pl.BlockSpec((F, d_out), lambda i: (0, 0)),
          pl.BlockSpec((F, d_out), lambda i: (0, 0)),
          pl.BlockSpec((1, d_out), lambda i: (0, 0)),
      ],
      out_specs=[
          pl.BlockSpec((_R, d_out), lambda i: (i, 0)),
          pl.BlockSpec((_R, d_out), lambda i: (i, 0)),
      ],
      out_shape=[jax.ShapeDtypeStruct((N, d_out), jnp.float32)] * 2,
  )(acc, deg, r_prev, W_l, W_r, b.reshape(1, d_out))


def _tc_final(acc, deg, r_prev):
  d = acc.shape[-1]

  def body(acc_ref, deg_ref, r_ref, o_ref):
    agg = acc_ref[0] + acc_ref[1]
    dg = deg_ref[0, :, 0:1] + deg_ref[1, :, 0:1]
    inv = 1.0 / jnp.maximum(dg, 1.0)
    o_ref[...] = agg * inv + r_ref[...]

  return pl.pallas_call(
      body,
      grid=(N // _R,),
      in_specs=[
          pl.BlockSpec((NC, _R, d), lambda i: (0, i, 0)),
          pl.BlockSpec((NC, _R, DEG_W), lambda i: (0, i, 0)),
          pl.BlockSpec((_R, d), lambda i: (i, 0)),
      ],
      out_specs=pl.BlockSpec((_R, d), lambda i: (i, 0)),
      out_shape=jax.ShapeDtypeStruct((N, d), jnp.float32),
  )(acc, deg, r_prev)


@jax.jit
def kernel(x, adjs, W_l1, W_r1, b1, W_l2, W_r2, b2, W_lo, W_ro, bo):
  src = adjs[0].astype(jnp.int32).reshape(E // CHUNK, CHUNK)
  dst = adjs[1].astype(jnp.int32).reshape(E // CHUNK, CHUNK)

  zrow128 = jnp.zeros((CHUNK, F), jnp.float32)
  zrow64 = jnp.zeros((CHUNK, C_OUT), jnp.float32)
  zdeg = jnp.zeros((CHUNK, DEG_W), jnp.float32)
  ones = jnp.ones((CHUNK, DEG_W), jnp.float32)

  sc_128_deg = _make_sc_segsum(F, with_deg=True)
  sc_128 = _make_sc_segsum(F, with_deg=False)
  sc_64 = _make_sc_segsum(C_OUT, with_deg=False)

  p1, r1 = _tc_in(x, W_l1, W_r1, b1)
  acc1, deg = sc_128_deg(p1, src, dst, zrow128, zdeg, ones)
  acc1 = acc1.reshape(NC, NPAD, F)
  deg = deg.reshape(NC, NPAD, DEG_W)
  p2, r2 = _tc_combine(acc1, deg, r1, W_l2, W_r2, b2)
  (acc2,) = sc_128(p2, src, dst, zrow128)
  acc2 = acc2.reshape(NC, NPAD, F)
  p3, r3 = _tc_combine(acc2, deg, r2, W_lo, W_ro, bo)
  (acc3,) = sc_64(p3, src, dst, zrow64)
  acc3 = acc3.reshape(NC, NPAD, C_OUT)
  return _tc_final(acc3, deg, r3)
